# Initial kernel scaffold; baseline (speedup 1.0000x reference)
import jax
import jax.numpy as jnp
from jax.experimental import pallas as pl


def kernel(x_user, x_group, x_place, x_event, x_item, edge_joins, edge_attends, edge_visits, edge_buys, edge_bought_by, params):
    raise NotImplementedError("write your pallas kernel here")


# speedup vs baseline: 13.8804x; 13.8804x over previous
import functools
import math

import jax
import jax.numpy as jnp
import numpy as np
from jax import lax
from jax.experimental import pallas as pl
from jax.experimental.pallas import tpu as pltpu
from jax.experimental.pallas import tpu_sc as plsc

_NT = ["user", "group", "place", "event", "item"]
_CNT = [100000, 20000, 20000, 30000, 50000]
_NOFF = [0, 100000, 120000, 140000, 170000]
_NTOT = 220000
_ET = [
    ("user", "joins", "group"),
    ("user", "attends", "event"),
    ("user", "visits", "place"),
    ("user", "buys", "item"),
    ("item", "bought_by", "user"),
]
_E = 100000
_HEADS, _DH, _HID = 4, 32, 128
_SQDH = math.sqrt(_DH)

_SRC_CNT = [100000, 100000, 100000, 100000, 50000]
_ROFF = [0, 100000, 200000, 300000, 400000]
_RTOT = 450000

_EPAD = 512000
_NW = 32
_PW = _EPAD // _NW
_GB = 128

_RC = 7680
_NCH = 30
_OPAD = _NCH * _RC
_TS = _RC // 16
_ZR = 24

_RB = 2000
_AB = 4000
_NEB = _EPAD // _AB
_NVE = 500000 // _AB

_F32 = jnp.float32


def _np_i32(x):
    return np.asarray(x, dtype=np.int32)


_TYPA = _np_i32(sum(([t] * (_CNT[t] // _RB) for t in range(5)), []))
_ETB = _np_i32(sum(([r] * (_SRC_CNT[r] // _RB) for r in range(5)), []))
_XB = _np_i32(
    sum(
        (
            [_NOFF[_NT.index(_ET[r][0])] // _RB + j for j in range(_SRC_CNT[r] // _RB)]
            for r in range(5)
        ),
        [],
    )
)
_EMAPS = np.stack([_ETB, _XB])

_S_HEAD = np.zeros((128, 16), np.float32)
for _h in range(4):
    _S_HEAD[_h * 32 : (_h + 1) * 32, _h] = 1.0
_B_HEAD = np.ascontiguousarray(_S_HEAD.T)


def _mm_typed_body(t_ref, x_ref, w_ref, b_ref, o_ref):
    o_ref[...] = (
        jnp.dot(x_ref[...], w_ref[0], preferred_element_type=_F32) + b_ref[0]
    )


def _mm_typed(x, w5, b5, out_cols):
    n = x.shape[0]
    gs = pltpu.PrefetchScalarGridSpec(
        num_scalar_prefetch=1,
        grid=(n // _RB,),
        in_specs=[
            pl.BlockSpec((_RB, 128), lambda i, t: (i, 0)),
            pl.BlockSpec((1, 128, out_cols), lambda i, t: (t[i], 0, 0)),
            pl.BlockSpec((1, 1, out_cols), lambda i, t: (t[i], 0, 0)),
        ],
        out_specs=pl.BlockSpec((_RB, out_cols), lambda i, t: (i, 0)),
    )
    return pl.pallas_call(
        _mm_typed_body,
        grid_spec=gs,
        out_shape=jax.ShapeDtypeStruct((n, out_cols), _F32),
    )(_TYPA, x, w5, b5.reshape(5, 1, out_cols))


def _mm_rel_body(e_ref, x_ref, w_ref, b_ref, a_ref, o_ref):
    tmp = jnp.dot(x_ref[...], w_ref[0], preferred_element_type=_F32) + b_ref[0]
    outs = []
    for h in range(_HEADS):
        outs.append(
            jnp.dot(
                tmp[:, h * _DH : (h + 1) * _DH],
                a_ref[0, h],
                preferred_element_type=_F32,
            )
        )
    o_ref[...] = jnp.concatenate(outs, axis=1)


def _mm_rel(h_all, w5, b5, a5):
    gs = pltpu.PrefetchScalarGridSpec(
        num_scalar_prefetch=1,
        grid=(_RTOT // _RB,),
        in_specs=[
            pl.BlockSpec((_RB, 128), lambda i, e: (e[1, i], 0)),
            pl.BlockSpec((1, 128, 128), lambda i, e: (e[0, i], 0, 0)),
            pl.BlockSpec((1, 1, 128), lambda i, e: (e[0, i], 0, 0)),
            pl.BlockSpec((1, _HEADS, _DH, _DH), lambda i, e: (e[0, i], 0, 0, 0)),
        ],
        out_specs=pl.BlockSpec((_RB, 128), lambda i, e: (i, 0)),
    )
    return pl.pallas_call(
        _mm_rel_body,
        grid_spec=gs,
        out_shape=jax.ShapeDtypeStruct((_RTOT, 128), _F32),
    )(_EMAPS, h_all, w5, b5.reshape(5, 1, 128), a5)


def _alpha_body(q_ref, k_ref, s_ref, a_ref, m_ref):
    al = jnp.dot(q_ref[...] * k_ref[...], s_ref[...], preferred_element_type=_F32)
    a_ref[...] = al
    m_ref[...] = jnp.max(al, axis=0, keepdims=True)[None]


def _alpha(q_dst, krel_src):
    return pl.pallas_call(
        _alpha_body,
        grid=(_NEB,),
        in_specs=[
            pl.BlockSpec((_AB, 128), lambda i: (i, 0)),
            pl.BlockSpec((_AB, 128), lambda i: (i, 0)),
            pl.BlockSpec((128, 16), lambda i: (0, 0)),
        ],
        out_specs=[
            pl.BlockSpec((_AB, 16), lambda i: (i, 0)),
            pl.BlockSpec((1, 1, 16), lambda i: (i, 0, 0)),
        ],
        out_shape=[
            jax.ShapeDtypeStruct((_EPAD, 16), _F32),
            jax.ShapeDtypeStruct((_NEB, 1, 16), _F32),
        ],
    )(q_dst, krel_src, _S_HEAD)


def _exmsg_body(a_ref, v_ref, m_ref, b_ref, ex_ref, msg_ref):
    m = jnp.max(m_ref[...], axis=0)
    e = jnp.exp(a_ref[...] - m)
    i = pl.program_id(0)
    e = jnp.where(i < _NVE, e, 0.0)
    ex_ref[...] = e
    msg_ref[...] = v_ref[...] * jnp.dot(e, b_ref[...], preferred_element_type=_F32)


def _exmsg(alpha, bmax, vrel_src):
    return pl.pallas_call(
        _exmsg_body,
        grid=(_NEB,),
        in_specs=[
            pl.BlockSpec((_AB, 16), lambda i: (i, 0)),
            pl.BlockSpec((_AB, 128), lambda i: (i, 0)),
            pl.BlockSpec((_NEB, 1, 16), lambda i: (0, 0, 0)),
            pl.BlockSpec((16, 128), lambda i: (0, 0)),
        ],
        out_specs=[
            pl.BlockSpec((_AB, 16), lambda i: (i, 0)),
            pl.BlockSpec((_AB, 128), lambda i: (i, 0)),
        ],
        out_shape=[
            jax.ShapeDtypeStruct((_EPAD, 16), _F32),
            jax.ShapeDtypeStruct((_EPAD, 128), _F32),
        ],
    )(alpha, vrel_src, bmax, _B_HEAD)


def _update_body(t_ref, o0_ref, o1_ref, d0_ref, d1_ref, h_ref, b16_ref, aw_ref,
                 ab_ref, sk_ref, out_ref):
    den = d0_ref[0] + d1_ref[0]
    denb = jnp.dot(den, b16_ref[...], preferred_element_type=_F32) + 1e-16
    an = (o0_ref[0] + o1_ref[0]) / denb
    g = jax.nn.gelu(an)
    o2 = jnp.dot(g, aw_ref[0], preferred_element_type=_F32) + ab_ref[0]
    beta = jax.nn.sigmoid(sk_ref[0])
    out_ref[...] = jnp.maximum(beta * o2 + (1.0 - beta) * h_ref[...], 0.0)


def _update(out_acc, den_acc, h_all, aw5, ab5, skip5):
    gs = pltpu.PrefetchScalarGridSpec(
        num_scalar_prefetch=1,
        grid=(_NTOT // _RB,),
        in_specs=[
            pl.BlockSpec((1, _RB, 128), lambda i, t: (0, i, 0)),
            pl.BlockSpec((1, _RB, 128), lambda i, t: (1, i, 0)),
            pl.BlockSpec((1, _RB, 16), lambda i, t: (0, i, 0)),
            pl.BlockSpec((1, _RB, 16), lambda i, t: (1, i, 0)),
            pl.BlockSpec((_RB, 128), lambda i, t: (i, 0)),
            pl.BlockSpec((16, 128), lambda i, t: (0, 0)),
            pl.BlockSpec((1, 128, 128), lambda i, t: (t[i], 0, 0)),
            pl.BlockSpec((1, 1, 128), lambda i, t: (t[i], 0, 0)),
            pl.BlockSpec((1, 1, 128), lambda i, t: (t[i], 0, 0)),
        ],
        out_specs=pl.BlockSpec((_RB, 128), lambda i, t: (i, 0)),
    )
    return pl.pallas_call(
        _update_body,
        grid_spec=gs,
        out_shape=jax.ShapeDtypeStruct((_NTOT, 128), _F32),
    )(_TYPA, out_acc, out_acc, den_acc, den_acc, h_all, _B_HEAD, aw5,
      ab5.reshape(5, 1, 128), skip5.reshape(5, 1, 128))


def _sc_mesh():
    return plsc.VectorSubcoreMesh(core_axis_name="c", subcore_axis_name="s")


_SC_PARAMS = pltpu.CompilerParams(needs_layout_passes=False, use_tc_tiling_on_sc=False)


def _gather3_body(qall, krel, vrel, gdst, gsrc, oq, ok, ov,
                  idxd, idxs, bq, bk, bv, semq, semk, semv):
    wid = lax.axis_index("s") * 2 + lax.axis_index("c")

    def body(j, _):
        base = wid * _PW + j * _GB
        pltpu.sync_copy(gdst.at[pl.ds(base, _GB)], idxd)
        pltpu.sync_copy(gsrc.at[pl.ds(base, _GB)], idxs)
        cq = pltpu.async_copy(qall.at[idxd], bq, semq)
        ck = pltpu.async_copy(krel.at[idxs], bk, semk)
        cv = pltpu.async_copy(vrel.at[idxs], bv, semv)
        cq.wait()
        pltpu.sync_copy(bq, oq.at[pl.ds(base, _GB)])
        ck.wait()
        pltpu.sync_copy(bk, ok.at[pl.ds(base, _GB)])
        cv.wait()
        pltpu.sync_copy(bv, ov.at[pl.ds(base, _GB)])
        return 0

    lax.fori_loop(0, _PW // _GB, body, 0)


def _sc_gather3(q_all, krel_all, vrel_all, gdst, gsrc):
    f = functools.partial(
        pl.kernel,
        out_type=[
            jax.ShapeDtypeStruct((_EPAD, 128), _F32),
            jax.ShapeDtypeStruct((_EPAD, 128), _F32),
            jax.ShapeDtypeStruct((_EPAD, 128), _F32),
        ],
        mesh=_sc_mesh(),
        compiler_params=_SC_PARAMS,
        scratch_types=[
            pltpu.VMEM((_GB,), jnp.int32),
            pltpu.VMEM((_GB,), jnp.int32),
            pltpu.VMEM((_GB, 128), _F32),
            pltpu.VMEM((_GB, 128), _F32),
            pltpu.VMEM((_GB, 128), _F32),
            pltpu.SemaphoreType.DMA,
            pltpu.SemaphoreType.DMA,
            pltpu.SemaphoreType.DMA,
        ],
    )
    return f(_gather3_body)(q_all, krel_all, vrel_all, gdst, gsrc)


def _scatter_body(msg, ex, gdst2, z128, z16, out_h, den_h,
                  dstv, wl, ubl, ube, mbuf, exbuf, zbuf, zbufd,
                  out_acc, den_acc, semm, seme):
    c = lax.axis_index("c")
    s = lax.axis_index("s")
    wid = s * 2 + c
    nrow = _PW // 16

    pltpu.sync_copy(gdst2.at[pl.ds(wid * nrow, nrow)], dstv)
    pltpu.sync_copy(z128, zbuf)
    pltpu.sync_copy(z16, zbufd)

    def prefix(v):
        iota16 = lax.iota(jnp.int32, 16)
        for sft in (1, 2, 4, 8):
            idx = jnp.maximum(iota16 - sft, 0)
            gv = v.at[idx].get(mode="promise_in_bounds")
            v = v + jnp.where(iota16 >= sft, gv, 0)
        return v

    for q in range(_NCH):
        base = q * _RC

        def zb(i, _):
            pltpu.sync_copy(zbuf, out_acc.at[pl.ds(s * _TS + i * _ZR, _ZR)])
            pltpu.sync_copy(zbufd, den_acc.at[pl.ds(s * _TS + i * _ZR, _ZR)])
            return 0

        lax.fori_loop(0, _TS // _ZR, zb, 0)
        plsc.subcore_barrier()

        def scan(g, cur):
            iota16 = lax.iota(jnp.int32, 16)
            d = dstv[g]
            loc = d - base
            m = (loc >= 0) & (loc < _RC)
            inc = prefix(m.astype(jnp.int32))
            offs = cur + inc - 1
            rows = lax.shift_right_logical(offs, 6)
            sub = jnp.bitwise_and(lax.shift_right_logical(offs, 4), 3)
            cols = jnp.bitwise_and(offs, 15)
            eid = wid * _PW + g * 16 + iota16
            packed = jnp.bitwise_or(lax.shift_left(loc, 19), eid)
            plsc.store_scatter(wl, [rows, sub, cols], packed, mask=m)
            return cur + plsc.all_reduce_population_count(m)

        cur = lax.fori_loop(0, nrow, scan, jnp.zeros((16,), jnp.int32))

        def padw(i, cur2):
            iota16 = lax.iota(jnp.int32, 16)
            offs = cur2 + iota16
            rows = lax.shift_right_logical(offs, 6)
            sub = jnp.bitwise_and(lax.shift_right_logical(offs, 4), 3)
            cols = jnp.bitwise_and(offs, 15)
            plsc.store_scatter(wl, [rows, sub, cols],
                               jnp.full((16,), 500000, jnp.int32))
            return cur2 + 16

        lax.fori_loop(0, 4, padw, cur)
        cnt = jnp.max(cur)
        nb = lax.div(cnt + 63, 64)

        def flush(b, _):
            for k in range(4):
                w = wl[b, k]
                ubl[pl.ds(k * 16, 16)] = jnp.bitwise_and(
                    lax.shift_right_logical(w, 19), 0x1FFF)
                ube[pl.ds(k * 16, 16)] = jnp.bitwise_and(w, 0x7FFFF)
            cm = pltpu.async_copy(msg.at[ube], mbuf, semm)
            ce = pltpu.async_copy(ex.at[ube], exbuf, seme)
            cm.wait()
            pltpu.sync_copy(mbuf, out_acc.at[ubl], add=True)
            ce.wait()
            pltpu.sync_copy(exbuf, den_acc.at[ubl], add=True)
            return 0

        lax.fori_loop(0, nb, flush, 0)
        plsc.subcore_barrier()

        pltpu.sync_copy(
            out_acc.at[pl.ds(s * _TS, _TS)],
            out_h.at[c, pl.ds(base + s * _TS, _TS)],
        )
        pltpu.sync_copy(
            den_acc.at[pl.ds(s * _TS, _TS)],
            den_h.at[c, pl.ds(base + s * _TS, _TS)],
        )
        plsc.subcore_barrier()


def _sc_scatter(msg, ex, gdst2, z128, z16):
    f = functools.partial(
        pl.kernel,
        out_type=[
            jax.ShapeDtypeStruct((2, _OPAD, 128), _F32),
            jax.ShapeDtypeStruct((2, _OPAD, 16), _F32),
        ],
        mesh=_sc_mesh(),
        compiler_params=_SC_PARAMS,
        scratch_types=[
            pltpu.VMEM((_PW // 16, 16), jnp.int32),
            pltpu.VMEM((252, 4, 16), jnp.int32),
            pltpu.VMEM((64,), jnp.int32),
            pltpu.VMEM((64,), jnp.int32),
            pltpu.VMEM((64, 128), _F32),
            pltpu.VMEM((64, 16), _F32),
            pltpu.VMEM((_ZR, 128), _F32),
            pltpu.VMEM((_ZR, 16), _F32),
            pltpu.VMEM_SHARED((_RC, 128), _F32),
            pltpu.VMEM_SHARED((_RC, 16), _F32),
            pltpu.SemaphoreType.DMA,
            pltpu.SemaphoreType.DMA,
        ],
    )
    return f(_scatter_body)(msg, ex, gdst2, z128, z16)


def kernel(x_user, x_group, x_place, x_event, x_item,
           edge_joins, edge_attends, edge_visits, edge_buys, edge_bought_by,
           params):
    xs = [x_user, x_group, x_place, x_event, x_item]
    edges = [edge_joins, edge_attends, edge_visits, edge_buys, edge_bought_by]

    x_all = jnp.concatenate(xs, axis=0)

    gsrc_parts, gdst_parts = [], []
    for r, (st, _, dt) in enumerate(_ET):
        ei = edges[r]
        gsrc_parts.append(ei[0] + _ROFF[r])
        gdst_parts.append(ei[1] + _NOFF[_NT.index(dt)])
    pad = jnp.zeros((_EPAD - 5 * _E,), jnp.int32)
    gsrc = jnp.concatenate(gsrc_parts + [pad])
    gdst = jnp.concatenate(gdst_parts + [pad])
    gdst2 = gdst.reshape(-1, 16)

    def stack_nt(d):
        return jnp.stack([d[nt] for nt in _NT])

    def stack_et(d):
        return jnp.stack([d[et[0]] for et in _ET])

    def stack_rel(d):
        return jnp.stack([d[et[1]] for et in _ET])

    lin_w5 = stack_nt(params["lin_W"])
    lin_b5 = stack_nt(params["lin_b"])
    out_w5 = jnp.pad(stack_nt(params["out_W"]), ((0, 0), (0, 0), (0, 64)))
    out_b5 = jnp.pad(stack_nt(params["out_b"]), ((0, 0), (0, 64)))

    z128 = jnp.zeros((_ZR, 128), _F32)
    z16 = jnp.zeros((_ZR, 16), _F32)

    h = _mm_typed(x_all, lin_w5, lin_b5, 128)

    for lp in params["convs"]:
        q_w5 = stack_nt(lp["q_W"])
        q_b5 = stack_nt(lp["q_b"])
        k_w5 = stack_et(lp["k_W"])
        k_b5 = stack_et(lp["k_b"])
        v_w5 = stack_et(lp["v_W"])
        v_b5 = stack_et(lp["v_b"])
        a_rel5 = stack_rel(lp["a_rel"])
        m_rel5 = stack_rel(lp["m_rel"])
        p_rel5 = stack_rel(lp["p_rel"])
        a_rel5 = a_rel5 * (p_rel5 / _SQDH)[:, :, None, None]
        aw5 = stack_nt(lp["a_W"])
        ab5 = stack_nt(lp["a_b"])
        skip5 = jnp.broadcast_to(jnp.stack([lp["skip"][nt] for nt in _NT])[:, None],
                                 (5, 128))

        q_all = _mm_typed(h, q_w5, q_b5, 128)
        krel = _mm_rel(h, k_w5, k_b5, a_rel5)
        vrel = _mm_rel(h, v_w5, v_b5, m_rel5)

        q_dst, krel_src, vrel_src = _sc_gather3(q_all, krel, vrel, gdst, gsrc)
        alpha, bmax = _alpha(q_dst, krel_src)
        ex, msg = _exmsg(alpha, bmax, vrel_src)
        out_acc, den_acc = _sc_scatter(msg, ex, gdst2, z128, z16)
        h = _update(out_acc, den_acc, h, aw5, ab5, skip5)

    y = _mm_typed(h, out_w5, out_b5, 128)
    outs = []
    for t in range(5):
        outs.append(y[_NOFF[t] : _NOFF[t] + _CNT[t], :64])
    return tuple(outs)
